# Initial kernel scaffold; baseline (speedup 1.0000x reference)
import jax
import jax.numpy as jnp
from jax.experimental import pallas as pl


def kernel(x, edge_index, W1, b1, W2, b2):
    raise NotImplementedError("write your pallas kernel here")


# speedup vs baseline: 8.5665x; 8.5665x over previous
import functools

import jax
import jax.numpy as jnp
from jax import lax
from jax.experimental import pallas as pl
from jax.experimental.pallas import tpu as pltpu
from jax.experimental.pallas import tpu_sc as plsc

N = 10000
NP = 10240
E = 320000
EP = 327680
D = 128
NSC = 2
NTEC = 16
ROWS_PER_TILE = NP // NTEC
ECHUNK = 128
CHUNKS_PER_TILE = EP // (NSC * NTEC * ECHUNK)
IDXROWS_PER_TILE_DEG = EP // (NTEC * ECHUNK)

def _deg_body(e_hbm, out_hbm, idx_b, zb, ones_b, deg_sh):
    c = lax.axis_index("c")
    s = lax.axis_index("s")
    zv = jnp.zeros((16,), jnp.float32)
    ov = jnp.ones((16,), jnp.float32)

    def fill_z(i, _):
        zb[pl.ds(i * 16, 16)] = zv
        return 0

    lax.fori_loop(0, ROWS_PER_TILE // 16, fill_z, 0)
    for i in range(ECHUNK // 16):
        ones_b[pl.ds(i * 16, 16)] = ov

    pltpu.sync_copy(zb, deg_sh.at[pl.ds(s * ROWS_PER_TILE, ROWS_PER_TILE)])
    plsc.subcore_barrier()

    pltpu.sync_copy(
        e_hbm.at[c, pl.ds(s * IDXROWS_PER_TILE_DEG, IDXROWS_PER_TILE_DEG)],
        idx_b,
    )

    def body(j, _):
        pltpu.sync_copy(ones_b, deg_sh.at[idx_b.at[j]], add=True)
        return 0

    lax.fori_loop(0, IDXROWS_PER_TILE_DEG, body, 0)
    plsc.subcore_barrier()
    pltpu.sync_copy(
        deg_sh.at[pl.ds(s * ROWS_PER_TILE, ROWS_PER_TILE)],
        out_hbm.at[pl.ds(c * NP + s * ROWS_PER_TILE, ROWS_PER_TILE)],
    )


def _agg_body(z_hbm, src_hbm, dst_hbm, out_hbm, sidx, didx, rba, rbb, zb, acc, gsa, gsb):
    c = lax.axis_index("c")
    s = lax.axis_index("s")
    wid = c * NTEC + s
    zv = jnp.zeros((16,), jnp.float32)

    def fill_z(r, _):
        for i in range(D // 16):
            zb[r, pl.ds(i * 16, 16)] = zv
        return 0

    lax.fori_loop(0, 16, fill_z, 0)

    def zero_acc(t, _):
        pltpu.sync_copy(zb, acc.at[pl.ds(s * ROWS_PER_TILE + t * 16, 16)])
        return 0

    lax.fori_loop(0, ROWS_PER_TILE // 16, zero_acc, 0)
    plsc.subcore_barrier()

    half = CHUNKS_PER_TILE // 2
    for h in range(2):
        pltpu.sync_copy(
            src_hbm.at[pl.ds(wid * CHUNKS_PER_TILE + h * half, half)], sidx)
        pltpu.sync_copy(
            dst_hbm.at[pl.ds(wid * CHUNKS_PER_TILE + h * half, half)], didx)

        def body(k, _):
            j0 = 2 * k
            j1 = 2 * k + 1
            ha = pltpu.async_copy(z_hbm.at[sidx.at[j0]], rba, gsa)
            hb = pltpu.async_copy(z_hbm.at[sidx.at[j1]], rbb, gsb)
            ha.wait()
            pltpu.sync_copy(rba, acc.at[didx.at[j0]], add=True)
            hb.wait()
            pltpu.sync_copy(rbb, acc.at[didx.at[j1]], add=True)
            return 0

        lax.fori_loop(0, half // 2, body, 0)
    plsc.subcore_barrier()
    pltpu.sync_copy(
        acc.at[pl.ds(s * ROWS_PER_TILE, ROWS_PER_TILE)],
        out_hbm.at[c, pl.ds(s * ROWS_PER_TILE, ROWS_PER_TILE)],
    )


@functools.cache
def _sc_kernels():
    mesh = plsc.VectorSubcoreMesh(core_axis_name="c", subcore_axis_name="s")
    deg = pl.kernel(
        _deg_body,
        mesh=mesh,
        out_type=jax.ShapeDtypeStruct((2 * NP,), jnp.float32),
        scratch_types=[
            pltpu.VMEM((IDXROWS_PER_TILE_DEG, ECHUNK), jnp.int32),
            pltpu.VMEM((ROWS_PER_TILE,), jnp.float32),
            pltpu.VMEM((ECHUNK,), jnp.float32),
            pltpu.VMEM_SHARED((NP,), jnp.float32),
        ],
    )
    agg = pl.kernel(
        _agg_body,
        mesh=mesh,
        out_type=jax.ShapeDtypeStruct((2, NP, D), jnp.float32),
        scratch_types=[
            pltpu.VMEM((CHUNKS_PER_TILE // 2, ECHUNK), jnp.int32),
            pltpu.VMEM((CHUNKS_PER_TILE // 2, ECHUNK), jnp.int32),
            pltpu.VMEM((ECHUNK, D), jnp.float32),
            pltpu.VMEM((ECHUNK, D), jnp.float32),
            pltpu.VMEM((16, D), jnp.float32),
            pltpu.VMEM_SHARED((NP, D), jnp.float32),
            pltpu.SemaphoreType.DMA,
            pltpu.SemaphoreType.DMA,
        ],
    )
    return deg, agg


def _norm_of(deg):
    return jnp.where(deg > 0, lax.rsqrt(jnp.maximum(deg, 1.0)), 0.0)


def _prep1_body(dego_ref, degi_ref, x_ref, w_ref, z_ref, on_ref, in_ref):
    on = _norm_of(dego_ref[...])
    inn = _norm_of(degi_ref[...])
    on_ref[...] = on
    in_ref[...] = inn
    z_ref[...] = jnp.dot(x_ref[...] * on, w_ref[...],
                         preferred_element_type=jnp.float32)


def _prep2_body(a_ref, b_ref, in_ref, on_ref, b1_ref, w_ref, z_ref):
    h = jnp.maximum(in_ref[...] * (a_ref[...] + b_ref[...]) + b1_ref[...], 0.0)
    z_ref[...] = jnp.dot(h * on_ref[...], w_ref[...],
                         preferred_element_type=jnp.float32)


def _finish_body(a_ref, b_ref, in_ref, b2_ref, o_ref):
    o_ref[...] = in_ref[...] * (a_ref[...] + b_ref[...]) + b2_ref[...]


_BR = 1024


def _col_spec(br):
    return pl.BlockSpec((br, 1), lambda i: (i, 0))


def _mat_spec(br):
    return pl.BlockSpec((br, D), lambda i: (i, 0))


_W_SPEC = pl.BlockSpec((D, D), lambda i: (0, 0))
_B_SPEC = pl.BlockSpec((1, D), lambda i: (0, 0))


def _prep1(dego, degi, x, w1):
    return pl.pallas_call(
        _prep1_body,
        grid=(NP // _BR,),
        in_specs=[_col_spec(_BR), _col_spec(_BR), _mat_spec(_BR), _W_SPEC],
        out_specs=[_mat_spec(_BR), _col_spec(_BR), _col_spec(_BR)],
        out_shape=[
            jax.ShapeDtypeStruct((NP, D), jnp.float32),
            jax.ShapeDtypeStruct((NP, 1), jnp.float32),
            jax.ShapeDtypeStruct((NP, 1), jnp.float32),
        ],
    )(dego, degi, x, w1)


def _prep2(a, b, inorm, onorm, b1, w2):
    return pl.pallas_call(
        _prep2_body,
        grid=(NP // _BR,),
        in_specs=[_mat_spec(_BR), _mat_spec(_BR), _col_spec(_BR),
                  _col_spec(_BR), _B_SPEC, _W_SPEC],
        out_specs=_mat_spec(_BR),
        out_shape=jax.ShapeDtypeStruct((NP, D), jnp.float32),
    )(a, b, inorm, onorm, b1, w2)


def _finish(a, b, inorm, b2):
    br = 1000
    return pl.pallas_call(
        _finish_body,
        grid=(N // br,),
        in_specs=[_mat_spec(br), _mat_spec(br), _col_spec(br), _B_SPEC],
        out_specs=_mat_spec(br),
        out_shape=jax.ShapeDtypeStruct((N, D), jnp.float32),
    )(a, b, inorm, b2)


def kernel(x, edge_index, W1, b1, W2, b2):
    ei = edge_index.astype(jnp.int32)
    pad_ids = N + (jnp.arange(EP - E, dtype=jnp.int32) % 128)
    ep = jnp.concatenate([ei, jnp.stack([pad_ids, pad_ids])], axis=1)
    e3d = ep.reshape(2, EP // ECHUNK, ECHUNK)
    src2d = e3d[0]
    dst2d = e3d[1]
    x_pad = jnp.pad(x, ((0, NP - N), (0, 0)))

    deg_k, agg_k = _sc_kernels()
    degflat = deg_k(e3d)
    dego = degflat[:NP].reshape(NP, 1)
    degi = degflat[NP:].reshape(NP, 1)

    z1, onorm, inorm = _prep1(dego, degi, x_pad, W1)
    agg1 = agg_k(z1, src2d, dst2d)
    z2 = _prep2(agg1[0], agg1[1], inorm, onorm, b1.reshape(1, D), W2)
    agg2 = agg_k(z2, src2d, dst2d)
    return _finish(agg2[0], agg2[1], inorm, b2.reshape(1, D))


# speedup vs baseline: 10.7839x; 1.2588x over previous
import functools

import jax
import jax.numpy as jnp
from jax import lax
from jax.experimental import pallas as pl
from jax.experimental.pallas import tpu as pltpu
from jax.experimental.pallas import tpu_sc as plsc

N = 10000
NP = 10240
E = 320000
EP = 327680
D = 128
NSC = 2
NTEC = 16
ROWS_PER_TILE = NP // NTEC
ECHUNK = 128
CHUNKS_PER_TILE = EP // (NSC * NTEC * ECHUNK)
IDXROWS_PER_TILE_DEG = EP // (NTEC * ECHUNK)

def _deg_body(e_hbm, out_hbm, idx_b, zb, ones_b, deg_sh):
    c = lax.axis_index("c")
    s = lax.axis_index("s")
    zv = jnp.zeros((16,), jnp.float32)
    ov = jnp.ones((16,), jnp.float32)

    def fill_z(i, _):
        zb[pl.ds(i * 16, 16)] = zv
        return 0

    lax.fori_loop(0, ROWS_PER_TILE // 16, fill_z, 0)
    for i in range(ECHUNK // 16):
        ones_b[pl.ds(i * 16, 16)] = ov

    pltpu.sync_copy(zb, deg_sh.at[pl.ds(s * ROWS_PER_TILE, ROWS_PER_TILE)])
    plsc.subcore_barrier()

    pltpu.sync_copy(
        e_hbm.at[c, pl.ds(s * IDXROWS_PER_TILE_DEG, IDXROWS_PER_TILE_DEG)],
        idx_b,
    )

    def body(j, _):
        pltpu.sync_copy(ones_b, deg_sh.at[idx_b.at[j]], add=True)
        return 0

    lax.fori_loop(0, IDXROWS_PER_TILE_DEG, body, 0)
    plsc.subcore_barrier()
    pltpu.sync_copy(
        deg_sh.at[pl.ds(s * ROWS_PER_TILE, ROWS_PER_TILE)],
        out_hbm.at[pl.ds(c * NP + s * ROWS_PER_TILE, ROWS_PER_TILE)],
    )


def _agg_body(z_hbm, src_hbm, dst_hbm, out_hbm, sidx, didx, rba, rbb, zb, acc, gsa, gsb):
    c = lax.axis_index("c")
    s = lax.axis_index("s")
    wid = c * NTEC + s
    zv = jnp.zeros((16,), jnp.float32)

    def fill_z(r, _):
        for i in range(D // 16):
            zb[r, pl.ds(i * 16, 16)] = zv
        return 0

    lax.fori_loop(0, 16, fill_z, 0)

    def zero_acc(t, _):
        pltpu.sync_copy(zb, acc.at[pl.ds(s * ROWS_PER_TILE + t * 16, 16)])
        return 0

    lax.fori_loop(0, ROWS_PER_TILE // 16, zero_acc, 0)
    plsc.subcore_barrier()

    half = CHUNKS_PER_TILE // 2
    for h in range(2):
        pltpu.sync_copy(
            src_hbm.at[pl.ds(wid * CHUNKS_PER_TILE + h * half, half)], sidx)
        pltpu.sync_copy(
            dst_hbm.at[pl.ds(wid * CHUNKS_PER_TILE + h * half, half)], didx)

        pltpu.async_copy(z_hbm.at[sidx.at[0]], rba, gsa)
        pltpu.async_copy(z_hbm.at[sidx.at[1]], rbb, gsb)

        def body(k, _):
            j0 = 2 * k
            j1 = 2 * k + 1
            pltpu.make_async_copy(z_hbm.at[sidx.at[j0]], rba, gsa).wait()
            pltpu.sync_copy(rba, acc.at[didx.at[j0]], add=True)

            @pl.when(j0 + 2 < half)
            def _():
                pltpu.async_copy(z_hbm.at[sidx.at[j0 + 2]], rba, gsa)

            pltpu.make_async_copy(z_hbm.at[sidx.at[j1]], rbb, gsb).wait()
            pltpu.sync_copy(rbb, acc.at[didx.at[j1]], add=True)

            @pl.when(j1 + 2 < half)
            def _():
                pltpu.async_copy(z_hbm.at[sidx.at[j1 + 2]], rbb, gsb)

            return 0

        lax.fori_loop(0, half // 2, body, 0)
    plsc.subcore_barrier()
    pltpu.sync_copy(
        acc.at[pl.ds(s * ROWS_PER_TILE, ROWS_PER_TILE)],
        out_hbm.at[c, pl.ds(s * ROWS_PER_TILE, ROWS_PER_TILE)],
    )


@functools.cache
def _sc_kernels():
    mesh = plsc.VectorSubcoreMesh(core_axis_name="c", subcore_axis_name="s")
    deg = pl.kernel(
        _deg_body,
        mesh=mesh,
        out_type=jax.ShapeDtypeStruct((2 * NP,), jnp.float32),
        scratch_types=[
            pltpu.VMEM((IDXROWS_PER_TILE_DEG, ECHUNK), jnp.int32),
            pltpu.VMEM((ROWS_PER_TILE,), jnp.float32),
            pltpu.VMEM((ECHUNK,), jnp.float32),
            pltpu.VMEM_SHARED((NP,), jnp.float32),
        ],
    )
    agg = pl.kernel(
        _agg_body,
        mesh=mesh,
        out_type=jax.ShapeDtypeStruct((2, NP, D), jnp.float32),
        scratch_types=[
            pltpu.VMEM((CHUNKS_PER_TILE // 2, ECHUNK), jnp.int32),
            pltpu.VMEM((CHUNKS_PER_TILE // 2, ECHUNK), jnp.int32),
            pltpu.VMEM((ECHUNK, D), jnp.float32),
            pltpu.VMEM((ECHUNK, D), jnp.float32),
            pltpu.VMEM((16, D), jnp.float32),
            pltpu.VMEM_SHARED((NP, D), jnp.float32),
            pltpu.SemaphoreType.DMA,
            pltpu.SemaphoreType.DMA,
        ],
    )
    return deg, agg


def _norm_of(deg):
    return jnp.where(deg > 0, lax.rsqrt(jnp.maximum(deg, 1.0)), 0.0)


def _prep1_body(dego_ref, degi_ref, x_ref, w_ref, z_ref, on_ref, in_ref):
    on = _norm_of(dego_ref[...])
    inn = _norm_of(degi_ref[...])
    on_ref[...] = on
    in_ref[...] = inn
    z_ref[...] = jnp.dot(x_ref[...] * on, w_ref[...],
                         preferred_element_type=jnp.float32)


def _prep2_body(a_ref, b_ref, in_ref, on_ref, b1_ref, w_ref, z_ref):
    h = jnp.maximum(in_ref[...] * (a_ref[...] + b_ref[...]) + b1_ref[...], 0.0)
    z_ref[...] = jnp.dot(h * on_ref[...], w_ref[...],
                         preferred_element_type=jnp.float32)


def _finish_body(a_ref, b_ref, in_ref, b2_ref, o_ref):
    o_ref[...] = in_ref[...] * (a_ref[...] + b_ref[...]) + b2_ref[...]


_BR = 1024


def _col_spec(br):
    return pl.BlockSpec((br, 1), lambda i: (i, 0))


def _mat_spec(br):
    return pl.BlockSpec((br, D), lambda i: (i, 0))


_W_SPEC = pl.BlockSpec((D, D), lambda i: (0, 0))
_B_SPEC = pl.BlockSpec((1, D), lambda i: (0, 0))


def _prep1(dego, degi, x, w1):
    return pl.pallas_call(
        _prep1_body,
        grid=(NP // _BR,),
        in_specs=[_col_spec(_BR), _col_spec(_BR), _mat_spec(_BR), _W_SPEC],
        out_specs=[_mat_spec(_BR), _col_spec(_BR), _col_spec(_BR)],
        out_shape=[
            jax.ShapeDtypeStruct((NP, D), jnp.float32),
            jax.ShapeDtypeStruct((NP, 1), jnp.float32),
            jax.ShapeDtypeStruct((NP, 1), jnp.float32),
        ],
    )(dego, degi, x, w1)


def _prep2(a, b, inorm, onorm, b1, w2):
    return pl.pallas_call(
        _prep2_body,
        grid=(NP // _BR,),
        in_specs=[_mat_spec(_BR), _mat_spec(_BR), _col_spec(_BR),
                  _col_spec(_BR), _B_SPEC, _W_SPEC],
        out_specs=_mat_spec(_BR),
        out_shape=jax.ShapeDtypeStruct((NP, D), jnp.float32),
    )(a, b, inorm, onorm, b1, w2)


def _finish(a, b, inorm, b2):
    br = 1000
    return pl.pallas_call(
        _finish_body,
        grid=(N // br,),
        in_specs=[_mat_spec(br), _mat_spec(br), _col_spec(br), _B_SPEC],
        out_specs=_mat_spec(br),
        out_shape=jax.ShapeDtypeStruct((N, D), jnp.float32),
    )(a, b, inorm, b2)


def kernel(x, edge_index, W1, b1, W2, b2):
    ei = edge_index.astype(jnp.int32)
    pad_ids = N + (jnp.arange(EP - E, dtype=jnp.int32) % 128)
    ep = jnp.concatenate([ei, jnp.stack([pad_ids, pad_ids])], axis=1)
    e3d = ep.reshape(2, EP // ECHUNK, ECHUNK)
    src2d = e3d[0]
    dst2d = e3d[1]
    x_pad = jnp.pad(x, ((0, NP - N), (0, 0)))

    deg_k, agg_k = _sc_kernels()
    degflat = deg_k(e3d)
    dego = degflat[:NP].reshape(NP, 1)
    degi = degflat[NP:].reshape(NP, 1)

    z1, onorm, inorm = _prep1(dego, degi, x_pad, W1)
    agg1 = agg_k(z1, src2d, dst2d)
    z2 = _prep2(agg1[0], agg1[1], inorm, onorm, b1.reshape(1, D), W2)
    agg2 = agg_k(z2, src2d, dst2d)
    return _finish(agg2[0], agg2[1], inorm, b2.reshape(1, D))


# speedup vs baseline: 11.4019x; 1.0573x over previous
import functools

import jax
import jax.numpy as jnp
from jax import lax
from jax.experimental import pallas as pl
from jax.experimental.pallas import tpu as pltpu
from jax.experimental.pallas import tpu_sc as plsc

N = 10000
NP = 10240
E = 320000
EP = 327680
D = 128
NSC = 2
NTEC = 16
ROWS_PER_TILE = NP // NTEC
ECHUNK = 128
EROWS = EP // ECHUNK
CHUNKS_PER_TILE = EROWS // (NSC * NTEC)
IDXROWS_PER_TILE_DEG = EROWS // NTEC


def _deg_body(src_hbm, dst_hbm, out_hbm, idx_b, zb, ones_b, deg_sh):
    c = lax.axis_index("c")
    s = lax.axis_index("s")
    zv = jnp.zeros((16,), jnp.float32)
    ov = jnp.ones((16,), jnp.float32)

    def fill_z(i, _):
        zb[pl.ds(i * 16, 16)] = zv
        return 0

    lax.fori_loop(0, ROWS_PER_TILE // 16, fill_z, 0)
    for i in range(ECHUNK // 16):
        ones_b[pl.ds(i * 16, 16)] = ov

    pltpu.sync_copy(zb, deg_sh.at[pl.ds(s * ROWS_PER_TILE, ROWS_PER_TILE)])
    plsc.subcore_barrier()

    @pl.when(c == 0)
    def _():
        pltpu.sync_copy(
            src_hbm.at[pl.ds(s * IDXROWS_PER_TILE_DEG, IDXROWS_PER_TILE_DEG)],
            idx_b)

    @pl.when(c == 1)
    def _():
        pltpu.sync_copy(
            dst_hbm.at[pl.ds(s * IDXROWS_PER_TILE_DEG, IDXROWS_PER_TILE_DEG)],
            idx_b)

    def body(j, _):
        pltpu.sync_copy(ones_b, deg_sh.at[idx_b.at[j]], add=True)
        return 0

    lax.fori_loop(0, IDXROWS_PER_TILE_DEG, body, 0)
    plsc.subcore_barrier()
    pltpu.sync_copy(
        deg_sh.at[pl.ds(s * ROWS_PER_TILE, ROWS_PER_TILE)],
        out_hbm.at[pl.ds(c * NP + s * ROWS_PER_TILE, ROWS_PER_TILE)],
    )


def _agg_body(z_hbm, src_hbm, dst_hbm, out0, out1, sidx, didx, rba, rbb, zb,
              acc, gsa, gsb):
    c = lax.axis_index("c")
    s = lax.axis_index("s")
    wid = c * NTEC + s
    zv = jnp.zeros((16,), jnp.float32)

    def fill_z(r, _):
        for i in range(D // 16):
            zb[r, pl.ds(i * 16, 16)] = zv
        return 0

    lax.fori_loop(0, 16, fill_z, 0)

    def zero_acc(t, _):
        pltpu.sync_copy(zb, acc.at[pl.ds(s * ROWS_PER_TILE + t * 16, 16)])
        return 0

    lax.fori_loop(0, ROWS_PER_TILE // 16, zero_acc, 0)
    plsc.subcore_barrier()

    half = CHUNKS_PER_TILE // 2
    for h in range(2):
        pltpu.sync_copy(
            src_hbm.at[pl.ds(wid * CHUNKS_PER_TILE + h * half, half)], sidx)
        pltpu.sync_copy(
            dst_hbm.at[pl.ds(wid * CHUNKS_PER_TILE + h * half, half)], didx)

        pltpu.async_copy(z_hbm.at[sidx.at[0]], rba, gsa)
        pltpu.async_copy(z_hbm.at[sidx.at[1]], rbb, gsb)

        def body(k, _):
            j0 = 2 * k
            j1 = 2 * k + 1
            pltpu.make_async_copy(z_hbm.at[sidx.at[j0]], rba, gsa).wait()
            pltpu.sync_copy(rba, acc.at[didx.at[j0]], add=True)

            @pl.when(j0 + 2 < half)
            def _():
                pltpu.async_copy(z_hbm.at[sidx.at[j0 + 2]], rba, gsa)

            pltpu.make_async_copy(z_hbm.at[sidx.at[j1]], rbb, gsb).wait()
            pltpu.sync_copy(rbb, acc.at[didx.at[j1]], add=True)

            @pl.when(j1 + 2 < half)
            def _():
                pltpu.async_copy(z_hbm.at[sidx.at[j1 + 2]], rbb, gsb)

            return 0

        lax.fori_loop(0, half // 2, body, 0)
    plsc.subcore_barrier()

    @pl.when(c == 0)
    def _():
        pltpu.sync_copy(acc.at[pl.ds(s * ROWS_PER_TILE, ROWS_PER_TILE)],
                        out0.at[pl.ds(s * ROWS_PER_TILE, ROWS_PER_TILE)])

    @pl.when(c == 1)
    def _():
        pltpu.sync_copy(acc.at[pl.ds(s * ROWS_PER_TILE, ROWS_PER_TILE)],
                        out1.at[pl.ds(s * ROWS_PER_TILE, ROWS_PER_TILE)])


@functools.cache
def _sc_kernels():
    mesh = plsc.VectorSubcoreMesh(core_axis_name="c", subcore_axis_name="s")
    deg = pl.kernel(
        _deg_body,
        mesh=mesh,
        out_type=jax.ShapeDtypeStruct((2 * NP,), jnp.float32),
        scratch_types=[
            pltpu.VMEM((IDXROWS_PER_TILE_DEG, ECHUNK), jnp.int32),
            pltpu.VMEM((ROWS_PER_TILE,), jnp.float32),
            pltpu.VMEM((ECHUNK,), jnp.float32),
            pltpu.VMEM_SHARED((NP,), jnp.float32),
        ],
    )
    agg = pl.kernel(
        _agg_body,
        mesh=mesh,
        out_type=[
            jax.ShapeDtypeStruct((NP, D), jnp.float32),
            jax.ShapeDtypeStruct((NP, D), jnp.float32),
        ],
        scratch_types=[
            pltpu.VMEM((CHUNKS_PER_TILE // 2, ECHUNK), jnp.int32),
            pltpu.VMEM((CHUNKS_PER_TILE // 2, ECHUNK), jnp.int32),
            pltpu.VMEM((ECHUNK, D), jnp.float32),
            pltpu.VMEM((ECHUNK, D), jnp.float32),
            pltpu.VMEM((16, D), jnp.float32),
            pltpu.VMEM_SHARED((NP, D), jnp.float32),
            pltpu.SemaphoreType.DMA,
            pltpu.SemaphoreType.DMA,
        ],
    )
    return deg, agg


def _norm_of(deg):
    return jnp.where(deg > 0, lax.rsqrt(jnp.maximum(deg, 1.0)), 0.0)


_BR = 1024
_BG = _BR // D


def _prep1_body(dego_ref, degi_ref, x_ref, w_ref, z_ref, on_ref, in_ref):
    on = _norm_of(dego_ref[...])
    inn = _norm_of(degi_ref[...])
    on_ref[...] = on
    in_ref[...] = inn
    xs = (x_ref[...] * on[:, :, None]).reshape(_BR, D)
    z_ref[...] = jnp.dot(xs, w_ref[...], preferred_element_type=jnp.float32)


def _prep2_body(a_ref, b_ref, in_ref, on_ref, b1_ref, w_ref, z_ref):
    h = jnp.maximum(
        in_ref[...][:, :, None] * (a_ref[...] + b_ref[...])
        + b1_ref[...].reshape(1, 1, D),
        0.0,
    )
    hs = (h * on_ref[...][:, :, None]).reshape(_BR, D)
    z_ref[...] = jnp.dot(hs, w_ref[...], preferred_element_type=jnp.float32)


def _finish_body(a_ref, b_ref, in_ref, b2_ref, o_ref):
    o = (in_ref[...][:, :, None] * (a_ref[...] + b_ref[...])
         + b2_ref[...].reshape(1, 1, D))
    o_ref[...] = o.reshape(_BR, D)


_V_SPEC = pl.BlockSpec((_BG, D), lambda i: (i, 0))
_M3_SPEC = pl.BlockSpec((_BG, D, D), lambda i: (i, 0, 0))
_M2_SPEC = pl.BlockSpec((_BR, D), lambda i: (i, 0))
_W_SPEC = pl.BlockSpec((D, D), lambda i: (0, 0))
_B_SPEC = pl.BlockSpec((1, D), lambda i: (0, 0))


def _prep1(dego, degi, x3, w1):
    return pl.pallas_call(
        _prep1_body,
        grid=(NP // _BR,),
        in_specs=[_V_SPEC, _V_SPEC, _M3_SPEC, _W_SPEC],
        out_specs=[_M2_SPEC, _V_SPEC, _V_SPEC],
        out_shape=[
            jax.ShapeDtypeStruct((NP, D), jnp.float32),
            jax.ShapeDtypeStruct((NP // D, D), jnp.float32),
            jax.ShapeDtypeStruct((NP // D, D), jnp.float32),
        ],
    )(dego, degi, x3, w1)


def _prep2(a3, b3, inorm, onorm, b1, w2):
    return pl.pallas_call(
        _prep2_body,
        grid=(NP // _BR,),
        in_specs=[_M3_SPEC, _M3_SPEC, _V_SPEC, _V_SPEC, _B_SPEC, _W_SPEC],
        out_specs=_M2_SPEC,
        out_shape=jax.ShapeDtypeStruct((NP, D), jnp.float32),
    )(a3, b3, inorm, onorm, b1, w2)


def _finish(a3, b3, inorm, b2):
    return pl.pallas_call(
        _finish_body,
        grid=(NP // _BR,),
        in_specs=[_M3_SPEC, _M3_SPEC, _V_SPEC, _B_SPEC],
        out_specs=_M2_SPEC,
        out_shape=jax.ShapeDtypeStruct((N, D), jnp.float32),
    )(a3, b3, inorm, b2)


def kernel(x, edge_index, W1, b1, W2, b2):
    ei = edge_index.astype(jnp.int32)
    pad_ids = N + (jnp.arange(EP - E, dtype=jnp.int32) % 128)
    srcp = jnp.concatenate([ei[0], pad_ids]).reshape(EROWS, ECHUNK)
    dstp = jnp.concatenate([ei[1], pad_ids]).reshape(EROWS, ECHUNK)
    x3 = jnp.pad(x, ((0, NP - N), (0, 0))).reshape(NP // D, D, D)

    deg_k, agg_k = _sc_kernels()
    degflat = deg_k(srcp, dstp)
    degmat = degflat.reshape(2 * NP // D, D)
    dego = degmat[: NP // D]
    degi = degmat[NP // D:]

    z1, onorm, inorm = _prep1(dego, degi, x3, W1)
    a1, b1_ = agg_k(z1, srcp, dstp)
    z2 = _prep2(a1.reshape(NP // D, D, D), b1_.reshape(NP // D, D, D),
                inorm, onorm, b1.reshape(1, D), W2)
    a2, b2_ = agg_k(z2, srcp, dstp)
    return _finish(a2.reshape(NP // D, D, D), b2_.reshape(NP // D, D, D),
                   inorm, b2.reshape(1, D))


# speedup vs baseline: 12.0427x; 1.0562x over previous
import functools

import jax
import jax.numpy as jnp
from jax import lax
from jax.experimental import pallas as pl
from jax.experimental.pallas import tpu as pltpu
from jax.experimental.pallas import tpu_sc as plsc

N = 10000
NP = 10240
E = 320000
EP = 327680
D = 128
NSC = 2
NTEC = 16
ROWS_PER_TILE = NP // NTEC
ECHUNK = 128
EROWS = EP // ECHUNK
CHUNKS_PER_TILE = EROWS // (NSC * NTEC)
IDXROWS_PER_TILE_DEG = EROWS // NTEC


def _deg_body(e_hbm, out_hbm, idx_b, zb, ones_b, deg_sh, *sems):
    c = lax.axis_index("c")
    s = lax.axis_index("s")
    zv = jnp.zeros((16,), jnp.float32)
    ov = jnp.ones((16,), jnp.float32)

    def fill_z(i, _):
        zb[pl.ds(i * 16, 16)] = zv
        return 0

    lax.fori_loop(0, ROWS_PER_TILE // 16, fill_z, 0)
    for i in range(ECHUNK // 16):
        ones_b[pl.ds(i * 16, 16)] = ov

    pltpu.sync_copy(zb, deg_sh.at[pl.ds(s * ROWS_PER_TILE, ROWS_PER_TILE)])
    plsc.subcore_barrier()

    pltpu.sync_copy(
        e_hbm.at[c, pl.ds(s * IDXROWS_PER_TILE_DEG, IDXROWS_PER_TILE_DEG)],
        idx_b)

    nd = len(sems)
    for b in range(nd):
        pltpu.async_copy(ones_b, deg_sh.at[idx_b.at[b]], sems[b], add=True)

    def body(k, _):
        for b in range(nd):
            j = nd * k + b
            pltpu.make_async_copy(ones_b, deg_sh.at[idx_b.at[j]],
                                  sems[b]).wait()

            @pl.when(j + nd < IDXROWS_PER_TILE_DEG)
            def _():
                pltpu.async_copy(ones_b, deg_sh.at[idx_b.at[j + nd]],
                                 sems[b], add=True)

        return 0

    lax.fori_loop(0, IDXROWS_PER_TILE_DEG // nd, body, 0)
    plsc.subcore_barrier()
    pltpu.sync_copy(
        deg_sh.at[pl.ds(s * ROWS_PER_TILE, ROWS_PER_TILE)],
        out_hbm.at[pl.ds(c * NP + s * ROWS_PER_TILE, ROWS_PER_TILE)],
    )


def _agg_body(z_hbm, e_hbm, out0, out1, sidx, didx, rba, rbb, zb,
              acc, gsa, gsb):
    c = lax.axis_index("c")
    s = lax.axis_index("s")
    wid = c * NTEC + s
    zv = jnp.zeros((16,), jnp.float32)

    def fill_z(r, _):
        for i in range(D // 16):
            zb[r, pl.ds(i * 16, 16)] = zv
        return 0

    lax.fori_loop(0, 16, fill_z, 0)

    def zero_acc(t, _):
        pltpu.sync_copy(zb, acc.at[pl.ds(s * ROWS_PER_TILE + t * 16, 16)])
        return 0

    lax.fori_loop(0, ROWS_PER_TILE // 16, zero_acc, 0)
    plsc.subcore_barrier()

    half = CHUNKS_PER_TILE // 2
    for h in range(2):
        pltpu.sync_copy(
            e_hbm.at[0, pl.ds(wid * CHUNKS_PER_TILE + h * half, half)], sidx)
        pltpu.sync_copy(
            e_hbm.at[1, pl.ds(wid * CHUNKS_PER_TILE + h * half, half)], didx)

        pltpu.async_copy(z_hbm.at[sidx.at[0]], rba, gsa)
        pltpu.async_copy(z_hbm.at[sidx.at[1]], rbb, gsb)

        def body(k, _):
            j0 = 2 * k
            j1 = 2 * k + 1
            pltpu.make_async_copy(z_hbm.at[sidx.at[j0]], rba, gsa).wait()
            pltpu.sync_copy(rba, acc.at[didx.at[j0]], add=True)

            @pl.when(j0 + 2 < half)
            def _():
                pltpu.async_copy(z_hbm.at[sidx.at[j0 + 2]], rba, gsa)

            pltpu.make_async_copy(z_hbm.at[sidx.at[j1]], rbb, gsb).wait()
            pltpu.sync_copy(rbb, acc.at[didx.at[j1]], add=True)

            @pl.when(j1 + 2 < half)
            def _():
                pltpu.async_copy(z_hbm.at[sidx.at[j1 + 2]], rbb, gsb)

            return 0

        lax.fori_loop(0, half // 2, body, 0)
    plsc.subcore_barrier()

    @pl.when(c == 0)
    def _():
        pltpu.sync_copy(acc.at[pl.ds(s * ROWS_PER_TILE, ROWS_PER_TILE)],
                        out0.at[pl.ds(s * ROWS_PER_TILE, ROWS_PER_TILE)])

    @pl.when(c == 1)
    def _():
        pltpu.sync_copy(acc.at[pl.ds(s * ROWS_PER_TILE, ROWS_PER_TILE)],
                        out1.at[pl.ds(s * ROWS_PER_TILE, ROWS_PER_TILE)])


@functools.cache
def _sc_kernels():
    mesh = plsc.VectorSubcoreMesh(core_axis_name="c", subcore_axis_name="s")
    deg = pl.kernel(
        _deg_body,
        mesh=mesh,
        out_type=jax.ShapeDtypeStruct((2 * NP,), jnp.float32),
        scratch_types=[
            pltpu.VMEM((IDXROWS_PER_TILE_DEG, ECHUNK), jnp.int32),
            pltpu.VMEM((ROWS_PER_TILE,), jnp.float32),
            pltpu.VMEM((ECHUNK,), jnp.float32),
            pltpu.VMEM_SHARED((NP,), jnp.float32),
            pltpu.SemaphoreType.DMA,
            pltpu.SemaphoreType.DMA,
            pltpu.SemaphoreType.DMA,
            pltpu.SemaphoreType.DMA,
        ],
    )
    agg = pl.kernel(
        _agg_body,
        mesh=mesh,
        out_type=[
            jax.ShapeDtypeStruct((NP, D), jnp.float32),
            jax.ShapeDtypeStruct((NP, D), jnp.float32),
        ],
        scratch_types=[
            pltpu.VMEM((CHUNKS_PER_TILE // 2, ECHUNK), jnp.int32),
            pltpu.VMEM((CHUNKS_PER_TILE // 2, ECHUNK), jnp.int32),
            pltpu.VMEM((ECHUNK, D), jnp.float32),
            pltpu.VMEM((ECHUNK, D), jnp.float32),
            pltpu.VMEM((16, D), jnp.float32),
            pltpu.VMEM_SHARED((NP, D), jnp.float32),
            pltpu.SemaphoreType.DMA,
            pltpu.SemaphoreType.DMA,
        ],
    )
    return deg, agg


def _norm_of(deg):
    return jnp.where(deg > 0, lax.rsqrt(jnp.maximum(deg, 1.0)), 0.0)


_BR = 1024
_BG = _BR // D


def _prep1_body(dego_ref, degi_ref, x_ref, w_ref, z_ref, on_ref, in_ref):
    on = _norm_of(dego_ref[...])
    inn = _norm_of(degi_ref[...])
    on_ref[...] = on
    in_ref[...] = inn
    xs = (x_ref[...] * on[:, :, None]).reshape(_BR, D)
    z_ref[...] = jnp.dot(xs, w_ref[...], preferred_element_type=jnp.float32)


def _prep2_body(a_ref, b_ref, in_ref, on_ref, b1_ref, w_ref, z_ref):
    h = jnp.maximum(
        in_ref[...][:, :, None] * (a_ref[...] + b_ref[...])
        + b1_ref[...].reshape(1, 1, D),
        0.0,
    )
    hs = (h * on_ref[...][:, :, None]).reshape(_BR, D)
    z_ref[...] = jnp.dot(hs, w_ref[...], preferred_element_type=jnp.float32)


def _finish_body(a_ref, b_ref, in_ref, b2_ref, o_ref):
    o = (in_ref[...][:, :, None] * (a_ref[...] + b_ref[...])
         + b2_ref[...].reshape(1, 1, D))
    o_ref[...] = o.reshape(_BR, D)


_V_SPEC = pl.BlockSpec((_BG, D), lambda i: (i, 0))
_M3_SPEC = pl.BlockSpec((_BG, D, D), lambda i: (i, 0, 0))
_M2_SPEC = pl.BlockSpec((_BR, D), lambda i: (i, 0))
_W_SPEC = pl.BlockSpec((D, D), lambda i: (0, 0))
_B_SPEC = pl.BlockSpec((1, D), lambda i: (0, 0))


def _prep1(dego, degi, x3, w1):
    return pl.pallas_call(
        _prep1_body,
        grid=(NP // _BR,),
        in_specs=[_V_SPEC, _V_SPEC, _M3_SPEC, _W_SPEC],
        out_specs=[_M2_SPEC, _V_SPEC, _V_SPEC],
        out_shape=[
            jax.ShapeDtypeStruct((NP, D), jnp.float32),
            jax.ShapeDtypeStruct((NP // D, D), jnp.float32),
            jax.ShapeDtypeStruct((NP // D, D), jnp.float32),
        ],
    )(dego, degi, x3, w1)


def _prep2(a3, b3, inorm, onorm, b1, w2):
    return pl.pallas_call(
        _prep2_body,
        grid=(NP // _BR,),
        in_specs=[_M3_SPEC, _M3_SPEC, _V_SPEC, _V_SPEC, _B_SPEC, _W_SPEC],
        out_specs=_M2_SPEC,
        out_shape=jax.ShapeDtypeStruct((NP, D), jnp.float32),
    )(a3, b3, inorm, onorm, b1, w2)


def _finish(a3, b3, inorm, b2):
    return pl.pallas_call(
        _finish_body,
        grid=(NP // _BR,),
        in_specs=[_M3_SPEC, _M3_SPEC, _V_SPEC, _B_SPEC],
        out_specs=_M2_SPEC,
        out_shape=jax.ShapeDtypeStruct((N, D), jnp.float32),
    )(a3, b3, inorm, b2)


def kernel(x, edge_index, W1, b1, W2, b2):
    ei = edge_index.astype(jnp.int32)
    pad_ids = N + (jnp.arange(EP - E, dtype=jnp.int32) % 128)
    pad2 = jnp.broadcast_to(pad_ids, (2, EP - E))
    ep3 = jnp.concatenate([ei, pad2], axis=1).reshape(2, EROWS, ECHUNK)
    x3 = jnp.pad(x, ((0, NP - N), (0, 0))).reshape(NP // D, D, D)

    deg_k, agg_k = _sc_kernels()
    degflat = deg_k(ep3)
    degmat = degflat.reshape(2 * NP // D, D)
    dego = degmat[: NP // D]
    degi = degmat[NP // D:]

    z1, onorm, inorm = _prep1(dego, degi, x3, W1)
    a1, b1_ = agg_k(z1, ep3)
    z2 = _prep2(a1.reshape(NP // D, D, D), b1_.reshape(NP // D, D, D),
                inorm, onorm, b1.reshape(1, D), W2)
    a2, b2_ = agg_k(z2, ep3)
    return _finish(a2.reshape(NP // D, D, D), b2_.reshape(NP // D, D, D),
                   inorm, b2.reshape(1, D))


# speedup vs baseline: 12.4552x; 1.0343x over previous
import functools

import jax
import jax.numpy as jnp
from jax import lax
from jax.experimental import pallas as pl
from jax.experimental.pallas import tpu as pltpu
from jax.experimental.pallas import tpu_sc as plsc

N = 10000
NP = 10240
E = 320000
EP = 327680
D = 128
NSC = 2
NTEC = 16
ROWS_PER_TILE = NP // NTEC
ECHUNK = 128
EROWS = EP // ECHUNK
CHUNKS_PER_TILE = EROWS // (NSC * NTEC)
IDXROWS_PER_TILE_DEG = EROWS // NTEC


def _deg_body(e_hbm, out_hbm, idx_b, zb, ones_b, deg_sh, *sems):
    c = lax.axis_index("c")
    s = lax.axis_index("s")
    zv = jnp.zeros((16,), jnp.float32)
    ov = jnp.ones((16,), jnp.float32)

    def fill_z(i, _):
        zb[pl.ds(i * 16, 16)] = zv
        return 0

    lax.fori_loop(0, ROWS_PER_TILE // 16, fill_z, 0)
    for i in range(ECHUNK // 16):
        ones_b[pl.ds(i * 16, 16)] = ov

    pltpu.sync_copy(zb, deg_sh.at[pl.ds(s * ROWS_PER_TILE, ROWS_PER_TILE)])
    plsc.subcore_barrier()

    pltpu.sync_copy(
        e_hbm.at[c, pl.ds(s * IDXROWS_PER_TILE_DEG, IDXROWS_PER_TILE_DEG)],
        idx_b)

    nd = len(sems)
    for b in range(nd):
        pltpu.async_copy(ones_b, deg_sh.at[idx_b.at[b]], sems[b], add=True)

    def body(k, _):
        for b in range(nd):
            j = nd * k + b
            pltpu.make_async_copy(ones_b, deg_sh.at[idx_b.at[j]],
                                  sems[b]).wait()

            @pl.when(j + nd < IDXROWS_PER_TILE_DEG)
            def _():
                pltpu.async_copy(ones_b, deg_sh.at[idx_b.at[j + nd]],
                                 sems[b], add=True)

        return 0

    lax.fori_loop(0, IDXROWS_PER_TILE_DEG // nd, body, 0)
    plsc.subcore_barrier()
    pltpu.sync_copy(
        deg_sh.at[pl.ds(s * ROWS_PER_TILE, ROWS_PER_TILE)],
        out_hbm.at[pl.ds(c * NP + s * ROWS_PER_TILE, ROWS_PER_TILE)],
    )


def _agg_body(z_hbm, e_hbm, out0, out1, sidx, didx, rba, rbb, zb,
              acc, gsa, gsb):
    c = lax.axis_index("c")
    s = lax.axis_index("s")
    wid = c * NTEC + s
    zv = jnp.zeros((16,), jnp.float32)

    half = CHUNKS_PER_TILE // 2

    pltpu.async_copy(
        e_hbm.at[0, pl.ds(wid * CHUNKS_PER_TILE, half)], sidx, gsa)
    pltpu.async_copy(
        e_hbm.at[1, pl.ds(wid * CHUNKS_PER_TILE, half)], didx, gsb)

    def fill_z(r, _):
        for i in range(D // 16):
            zb[r, pl.ds(i * 16, 16)] = zv
        return 0

    lax.fori_loop(0, 16, fill_z, 0)

    def zero_acc(t, _):
        pltpu.sync_copy(zb, acc.at[pl.ds(s * ROWS_PER_TILE + t * 16, 16)])
        return 0

    lax.fori_loop(0, ROWS_PER_TILE // 16, zero_acc, 0)
    plsc.subcore_barrier()

    for h in range(2):
        if h == 0:
            pltpu.make_async_copy(
                e_hbm.at[0, pl.ds(wid * CHUNKS_PER_TILE, half)], sidx,
                gsa).wait()
            pltpu.make_async_copy(
                e_hbm.at[1, pl.ds(wid * CHUNKS_PER_TILE, half)], didx,
                gsb).wait()
        else:
            pltpu.sync_copy(
                e_hbm.at[0, pl.ds(wid * CHUNKS_PER_TILE + h * half, half)],
                sidx)
            pltpu.sync_copy(
                e_hbm.at[1, pl.ds(wid * CHUNKS_PER_TILE + h * half, half)],
                didx)

        pltpu.async_copy(z_hbm.at[sidx.at[0]], rba, gsa)
        pltpu.async_copy(z_hbm.at[sidx.at[1]], rbb, gsb)

        def body(k, _):
            j0 = 2 * k
            j1 = 2 * k + 1
            pltpu.make_async_copy(z_hbm.at[sidx.at[j0]], rba, gsa).wait()
            pltpu.sync_copy(rba, acc.at[didx.at[j0]], add=True)

            @pl.when(j0 + 2 < half)
            def _():
                pltpu.async_copy(z_hbm.at[sidx.at[j0 + 2]], rba, gsa)

            pltpu.make_async_copy(z_hbm.at[sidx.at[j1]], rbb, gsb).wait()
            pltpu.sync_copy(rbb, acc.at[didx.at[j1]], add=True)

            @pl.when(j1 + 2 < half)
            def _():
                pltpu.async_copy(z_hbm.at[sidx.at[j1 + 2]], rbb, gsb)

            return 0

        lax.fori_loop(0, half // 2, body, 0)
    plsc.subcore_barrier()

    @pl.when(c == 0)
    def _():
        pltpu.sync_copy(acc.at[pl.ds(s * ROWS_PER_TILE, ROWS_PER_TILE)],
                        out0.at[pl.ds(s * ROWS_PER_TILE, ROWS_PER_TILE)])

    @pl.when(c == 1)
    def _():
        pltpu.sync_copy(acc.at[pl.ds(s * ROWS_PER_TILE, ROWS_PER_TILE)],
                        out1.at[pl.ds(s * ROWS_PER_TILE, ROWS_PER_TILE)])


@functools.cache
def _sc_kernels():
    mesh = plsc.VectorSubcoreMesh(core_axis_name="c", subcore_axis_name="s")
    deg = pl.kernel(
        _deg_body,
        mesh=mesh,
        out_type=jax.ShapeDtypeStruct((2 * NP,), jnp.float32),
        scratch_types=[
            pltpu.VMEM((IDXROWS_PER_TILE_DEG, ECHUNK), jnp.int32),
            pltpu.VMEM((ROWS_PER_TILE,), jnp.float32),
            pltpu.VMEM((ECHUNK,), jnp.float32),
            pltpu.VMEM_SHARED((NP,), jnp.float32),
            pltpu.SemaphoreType.DMA,
            pltpu.SemaphoreType.DMA,
            pltpu.SemaphoreType.DMA,
            pltpu.SemaphoreType.DMA,
        ],
    )
    agg = pl.kernel(
        _agg_body,
        mesh=mesh,
        out_type=[
            jax.ShapeDtypeStruct((NP, D), jnp.float32),
            jax.ShapeDtypeStruct((NP, D), jnp.float32),
        ],
        scratch_types=[
            pltpu.VMEM((CHUNKS_PER_TILE // 2, ECHUNK), jnp.int32),
            pltpu.VMEM((CHUNKS_PER_TILE // 2, ECHUNK), jnp.int32),
            pltpu.VMEM((ECHUNK, D), jnp.float32),
            pltpu.VMEM((ECHUNK, D), jnp.float32),
            pltpu.VMEM((16, D), jnp.float32),
            pltpu.VMEM_SHARED((NP, D), jnp.float32),
            pltpu.SemaphoreType.DMA,
            pltpu.SemaphoreType.DMA,
        ],
    )
    return deg, agg


def _norm_of(deg):
    return jnp.where(deg > 0, lax.rsqrt(jnp.maximum(deg, 1.0)), 0.0)


_BR = 2048
_BG = _BR // D


def _prep1_body(dego_ref, degi_ref, x_ref, w_ref, z_ref, on_ref, in_ref):
    on = _norm_of(dego_ref[...])
    inn = _norm_of(degi_ref[...])
    on_ref[...] = on
    in_ref[...] = inn
    xs = (x_ref[...] * on[:, :, None]).reshape(_BR, D)
    z_ref[...] = jnp.dot(xs, w_ref[...], preferred_element_type=jnp.float32)


def _prep2_body(a_ref, b_ref, in_ref, on_ref, b1_ref, w_ref, z_ref):
    h = jnp.maximum(
        in_ref[...][:, :, None] * (a_ref[...] + b_ref[...])
        + b1_ref[...].reshape(1, 1, D),
        0.0,
    )
    hs = (h * on_ref[...][:, :, None]).reshape(_BR, D)
    z_ref[...] = jnp.dot(hs, w_ref[...], preferred_element_type=jnp.float32)


def _finish_body(a_ref, b_ref, in_ref, b2_ref, o_ref):
    o = (in_ref[...][:, :, None] * (a_ref[...] + b_ref[...])
         + b2_ref[...].reshape(1, 1, D))
    o_ref[...] = o.reshape(_BR, D)


_V_SPEC = pl.BlockSpec((_BG, D), lambda i: (i, 0))
_M3_SPEC = pl.BlockSpec((_BG, D, D), lambda i: (i, 0, 0))
_M2_SPEC = pl.BlockSpec((_BR, D), lambda i: (i, 0))
_W_SPEC = pl.BlockSpec((D, D), lambda i: (0, 0))
_B_SPEC = pl.BlockSpec((1, D), lambda i: (0, 0))


def _prep1(dego, degi, x3, w1):
    return pl.pallas_call(
        _prep1_body,
        grid=(NP // _BR,),
        in_specs=[_V_SPEC, _V_SPEC, _M3_SPEC, _W_SPEC],
        out_specs=[_M2_SPEC, _V_SPEC, _V_SPEC],
        out_shape=[
            jax.ShapeDtypeStruct((NP, D), jnp.float32),
            jax.ShapeDtypeStruct((NP // D, D), jnp.float32),
            jax.ShapeDtypeStruct((NP // D, D), jnp.float32),
        ],
    )(dego, degi, x3, w1)


def _prep2(a3, b3, inorm, onorm, b1, w2):
    return pl.pallas_call(
        _prep2_body,
        grid=(NP // _BR,),
        in_specs=[_M3_SPEC, _M3_SPEC, _V_SPEC, _V_SPEC, _B_SPEC, _W_SPEC],
        out_specs=_M2_SPEC,
        out_shape=jax.ShapeDtypeStruct((NP, D), jnp.float32),
    )(a3, b3, inorm, onorm, b1, w2)


def _finish(a3, b3, inorm, b2):
    return pl.pallas_call(
        _finish_body,
        grid=(NP // _BR,),
        in_specs=[_M3_SPEC, _M3_SPEC, _V_SPEC, _B_SPEC],
        out_specs=_M2_SPEC,
        out_shape=jax.ShapeDtypeStruct((N, D), jnp.float32),
    )(a3, b3, inorm, b2)


def kernel(x, edge_index, W1, b1, W2, b2):
    ei = edge_index.astype(jnp.int32)
    pad_ids = N + (jnp.arange(EP - E, dtype=jnp.int32) % 128)
    pad2 = jnp.broadcast_to(pad_ids, (2, EP - E))
    ep3 = jnp.concatenate([ei, pad2], axis=1).reshape(2, EROWS, ECHUNK)
    x3 = jnp.pad(x, ((0, NP - N), (0, 0))).reshape(NP // D, D, D)

    deg_k, agg_k = _sc_kernels()
    degflat = deg_k(ep3)
    degmat = degflat.reshape(2 * NP // D, D)
    dego = degmat[: NP // D]
    degi = degmat[NP // D:]

    z1, onorm, inorm = _prep1(dego, degi, x3, W1)
    a1, b1_ = agg_k(z1, ep3)
    z2 = _prep2(a1.reshape(NP // D, D, D), b1_.reshape(NP // D, D, D),
                inorm, onorm, b1.reshape(1, D), W2)
    a2, b2_ = agg_k(z2, ep3)
    return _finish(a2.reshape(NP // D, D, D), b2_.reshape(NP // D, D, D),
                   inorm, b2.reshape(1, D))
